# baseline (speedup 1.0000x reference)
import functools

import jax
import jax.numpy as jnp
from jax import lax
from jax.experimental import pallas as pl
from jax.experimental.pallas import tpu as pltpu
from jax.experimental.pallas import tpu_sc as plsc

GTF = 3
L = 16
NC, NS = 2, 16
NW = NC * NS
CH = 128


def _make_sc_kernel(B, S, P):
    N = B * S
    assert N % NW == 0
    per_w = N // NW
    assert per_w % CH == 0
    nch = per_w // CH
    NR = (B * P * GTF * 3) // L
    assert NR * L == B * P * GTF * 3

    mesh = plsc.VectorSubcoreMesh(core_axis_name="c", subcore_axis_name="s")

    @functools.partial(
        pl.kernel,
        mesh=mesh,
        compiler_params=pltpu.CompilerParams(
            needs_layout_passes=False, use_tc_tiling_on_sc=False),
        out_type=[
            jax.ShapeDtypeStruct((NW, L), jnp.float32),
            jax.ShapeDtypeStruct((NW, L), jnp.float32),
        ],
        scratch_types=[
            pltpu.VMEM((nch, CH), jnp.int32),
            pltpu.VMEM((nch, CH), jnp.int32),
            pltpu.VMEM((nch, CH), jnp.int32),
            pltpu.VMEM((per_w, L), jnp.float32),
            pltpu.VMEM((per_w, L), jnp.float32),
            pltpu.VMEM((per_w,), jnp.int32),
            pltpu.VMEM((per_w, 3), jnp.float32),
            pltpu.VMEM((per_w, 3), jnp.float32),
            pltpu.VMEM((L,), jnp.float32),
            pltpu.VMEM((L,), jnp.float32),
            pltpu.SemaphoreType.DMA,
        ],
    )
    def sc_kernel(idx_hbm, seed_hbm, vote_hbm, vl_hbm, mask_hbm,
                  num_hbm, den_hbm,
                  idx_v, j0_v, j1_v, s0_v, s1_v, mask_v, seed_v, vote_v,
                  accn_v, accd_v, sem):
        cid = lax.axis_index("c")
        sid = lax.axis_index("s")
        wid = sid * NC + cid
        base = wid * per_w

        pltpu.sync_copy(idx_hbm.at[pl.ds(wid * nch, nch)], idx_v)
        pltpu.sync_copy(seed_hbm.at[pl.ds(base, per_w)], seed_v)
        pltpu.sync_copy(vote_hbm.at[pl.ds(base, per_w)], vote_v)

        for c in range(nch):
            for s in range(CH // L):
                idx16 = idx_v[c, pl.ds(s * L, L)]
                j0 = lax.shift_right_logical(idx16 * 9, 4)
                j0_v[c, pl.ds(s * L, L)] = j0
                j1_v[c, pl.ds(s * L, L)] = jnp.minimum(j0 + 1, NR - 1)

        descs = []
        for c in range(nch):
            dst = pl.ds(c * CH, CH)
            descs.append(pltpu.async_copy(
                vl_hbm.at[j0_v.at[c]], s0_v.at[dst], sem))
            descs.append(pltpu.async_copy(
                vl_hbm.at[j1_v.at[c]], s1_v.at[dst], sem))
            descs.append(pltpu.async_copy(
                mask_hbm.at[idx_v.at[c]], mask_v.at[dst], sem))
        for dsc in descs:
            dsc.wait()

        iota = lax.iota(jnp.int32, L)
        zeros = jnp.zeros((L,), jnp.float32)

        def chunk_body(c, g, carry):
            num, den = carry
            ridx = c * CH + g * L + iota
            idx16 = idx_v[c, pl.ds(g * L, L)]
            o = jnp.bitwise_and(idx16 * 9, L - 1)

            def gat(ref, col):
                cidx = jnp.full((L,), col, jnp.int32)
                return plsc.load_gather(ref, [ridx, cidx])

            sx = [gat(seed_v, k) for k in range(3)]
            vx = [gat(vote_v, k) for k in range(3)]

            d = None
            for j in range(GTF):
                dj = None
                for k in range(3):
                    pos = o + (3 * j + k)
                    col = jnp.bitwise_and(pos, L - 1)
                    a = plsc.load_gather(s0_v, [ridx, col])
                    b = plsc.load_gather(s1_v, [ridx, col])
                    gt = jnp.where(pos < L, a, b)
                    t = jnp.abs(vx[k] - (gt + sx[k]))
                    dj = t if dj is None else dj + t
                d = dj if d is None else jnp.minimum(d, dj)
            mf = mask_v[pl.ds(c * CH + g * L, L)].astype(jnp.float32)
            return num + d * mf, den + mf

        acc = (zeros, zeros)
        for c in range(nch):
            acc = lax.fori_loop(
                0, CH // L, functools.partial(chunk_body, c), acc)
        num, den = acc

        accn_v[...] = num
        accd_v[...] = den
        pltpu.sync_copy(accn_v, num_hbm.at[wid])
        pltpu.sync_copy(accd_v, den_hbm.at[wid])

    return sc_kernel


def kernel(seed_xyz, vote_xyz, seed_inds, vote_label_mask, vote_label):
    B, S, _ = seed_xyz.shape
    P = vote_label.shape[1]
    N = B * S

    idx_g = (seed_inds.astype(jnp.int32)
             + (jnp.arange(B, dtype=jnp.int32) * P)[:, None])
    idx_g = idx_g.reshape(N // CH, CH)
    seed_flat = seed_xyz.reshape(N, 3)
    vote_flat = vote_xyz.reshape(N, 3)
    vl_gran = vote_label.reshape((B * P * GTF * 3) // L, L)
    mask_flat = vote_label_mask.astype(jnp.int32).reshape(B * P)

    sc = _make_sc_kernel(B, S, P)
    num, den = sc(idx_g, seed_flat, vote_flat, vl_gran, mask_flat)
    return jnp.sum(num) / (jnp.sum(den) + 1e-6)


# speedup vs baseline: 5.7005x; 5.7005x over previous
import functools

import jax
import jax.numpy as jnp
from jax import lax
from jax.experimental import pallas as pl
from jax.experimental.pallas import tpu as pltpu
from jax.experimental.pallas import tpu_sc as plsc

GTF = 3
NCOMP = GTF * 3
L = 16
NC, NS = 2, 16
NW = NC * NS
CH = 128


def _make_sc_kernel(B, S, P):
    N = B * S
    assert N % NW == 0
    per_w = N // NW
    assert per_w % CH == 0
    nch = per_w // CH
    BP = B * P

    mesh = plsc.VectorSubcoreMesh(core_axis_name="c", subcore_axis_name="s")

    @functools.partial(
        pl.kernel,
        mesh=mesh,
        compiler_params=pltpu.CompilerParams(
            needs_layout_passes=False, use_tc_tiling_on_sc=False),
        out_type=[
            jax.ShapeDtypeStruct((NW, L), jnp.float32),
            jax.ShapeDtypeStruct((NW, L), jnp.float32),
        ],
        scratch_types=[
            pltpu.VMEM((nch, CH), jnp.int32),
            pltpu.VMEM((NCOMP * nch, CH), jnp.int32),
            pltpu.VMEM((NCOMP, per_w), jnp.float32),
            pltpu.VMEM((per_w,), jnp.int32),
            pltpu.VMEM((3, per_w), jnp.float32),
            pltpu.VMEM((3, per_w), jnp.float32),
            pltpu.VMEM((L,), jnp.float32),
            pltpu.VMEM((L,), jnp.float32),
            pltpu.SemaphoreType.DMA,
        ],
    )
    def sc_kernel(idx_hbm, seed_hbm, vote_hbm, vl_hbm, mask_hbm,
                  num_hbm, den_hbm,
                  idx_v, idx9_v, gt_v, mask_v, seed_v, vote_v,
                  accn_v, accd_v, sem):
        cid = lax.axis_index("c")
        sid = lax.axis_index("s")
        wid = sid * NC + cid
        base = wid * per_w

        pltpu.sync_copy(idx_hbm.at[pl.ds(wid * nch, nch)], idx_v)
        for k in range(3):
            pltpu.sync_copy(seed_hbm.at[k, pl.ds(base, per_w)], seed_v.at[k])
            pltpu.sync_copy(vote_hbm.at[k, pl.ds(base, per_w)], vote_v.at[k])

        for c in range(nch):
            for s in range(CH // L):
                idx16 = idx_v[c, pl.ds(s * L, L)]
                for k in range(NCOMP):
                    idx9_v[k * nch + c, pl.ds(s * L, L)] = idx16 + k * BP

        descs = []
        for c in range(nch):
            dst = pl.ds(c * CH, CH)
            descs.append(pltpu.async_copy(
                mask_hbm.at[idx_v.at[c]], mask_v.at[dst], sem))
            for k in range(NCOMP):
                descs.append(pltpu.async_copy(
                    vl_hbm.at[idx9_v.at[k * nch + c]], gt_v.at[k, dst], sem))
        for dsc in descs:
            dsc.wait()

        zeros = jnp.zeros((L,), jnp.float32)

        def body(g, carry):
            num, den = carry
            sl = pl.ds(g * L, L)
            sx = [seed_v[k, sl] for k in range(3)]
            vx = [vote_v[k, sl] for k in range(3)]
            d = None
            for j in range(GTF):
                dj = None
                for k in range(3):
                    t = jnp.abs(vx[k] - (gt_v[3 * j + k, sl] + sx[k]))
                    dj = t if dj is None else dj + t
                d = dj if d is None else jnp.minimum(d, dj)
            mf = mask_v[sl].astype(jnp.float32)
            return num + d * mf, den + mf

        num, den = lax.fori_loop(0, per_w // L, body, (zeros, zeros))

        accn_v[...] = num
        accd_v[...] = den
        pltpu.sync_copy(accn_v, num_hbm.at[wid])
        pltpu.sync_copy(accd_v, den_hbm.at[wid])

    return sc_kernel


def kernel(seed_xyz, vote_xyz, seed_inds, vote_label_mask, vote_label):
    B, S, _ = seed_xyz.shape
    P = vote_label.shape[1]
    N = B * S

    idx_g = (seed_inds.astype(jnp.int32)
             + (jnp.arange(B, dtype=jnp.int32) * P)[:, None])
    idx_g = idx_g.reshape(N // CH, CH)
    seed_t = jnp.transpose(seed_xyz, (2, 0, 1)).reshape(3, N)
    vote_t = jnp.transpose(vote_xyz, (2, 0, 1)).reshape(3, N)
    vl_t = jnp.transpose(vote_label, (2, 0, 1)).reshape(NCOMP * B * P)
    mask_flat = vote_label_mask.astype(jnp.int32).reshape(B * P)

    sc = _make_sc_kernel(B, S, P)
    num, den = sc(idx_g, seed_t, vote_t, vl_t, mask_flat)
    return jnp.sum(num) / (jnp.sum(den) + 1e-6)


# speedup vs baseline: 6.0090x; 1.0541x over previous
import functools

import jax
import jax.numpy as jnp
from jax import lax
from jax.experimental import pallas as pl
from jax.experimental.pallas import tpu as pltpu
from jax.experimental.pallas import tpu_sc as plsc

GTF = 3
NCOMP = GTF * 3
L = 16
NC, NS = 2, 16
NW = NC * NS
CH = 128


def _make_sc_kernel(B, S, P):
    N = B * S
    assert N % NW == 0
    per_w = N // NW
    assert per_w % CH == 0
    nch = per_w // CH
    BP = B * P

    mesh = plsc.VectorSubcoreMesh(core_axis_name="c", subcore_axis_name="s")

    @functools.partial(
        pl.kernel,
        mesh=mesh,
        compiler_params=pltpu.CompilerParams(
            needs_layout_passes=False, use_tc_tiling_on_sc=False),
        out_type=[
            jax.ShapeDtypeStruct((NW, L), jnp.float32),
            jax.ShapeDtypeStruct((NW, L), jnp.float32),
        ],
        scratch_types=[
            pltpu.VMEM((nch, CH), jnp.int32),
            pltpu.VMEM((NCOMP * nch, CH), jnp.int32),
            pltpu.VMEM((NCOMP, per_w), jnp.float32),
            pltpu.VMEM((per_w,), jnp.int32),
            pltpu.VMEM((3, per_w), jnp.float32),
            pltpu.VMEM((3, per_w), jnp.float32),
            pltpu.VMEM((L,), jnp.float32),
            pltpu.VMEM((L,), jnp.float32),
            pltpu.SemaphoreType.DMA,
            pltpu.SemaphoreType.DMA,
        ],
    )
    def sc_kernel(idx_hbm, seed_hbm, vote_hbm, vl_hbm, mask_hbm,
                  num_hbm, den_hbm,
                  idx_v, idx9_v, gt_v, mask_v, seed_v, vote_v,
                  accn_v, accd_v, sem, xyz_sem):
        cid = lax.axis_index("c")
        sid = lax.axis_index("s")
        wid = sid * NC + cid
        base = wid * per_w

        pltpu.sync_copy(idx_hbm.at[pl.ds(wid * nch, nch)], idx_v)
        xyz_descs = []
        for k in range(3):
            xyz_descs.append(pltpu.async_copy(
                seed_hbm.at[k, pl.ds(base, per_w)], seed_v.at[k], xyz_sem))
            xyz_descs.append(pltpu.async_copy(
                vote_hbm.at[k, pl.ds(base, per_w)], vote_v.at[k], xyz_sem))

        descs = []
        for c in range(nch):
            for s in range(CH // L):
                idx16 = idx_v[c, pl.ds(s * L, L)]
                for k in range(NCOMP):
                    idx9_v[k * nch + c, pl.ds(s * L, L)] = idx16 + k * BP
            dst = pl.ds(c * CH, CH)
            chunk_descs = [pltpu.async_copy(
                mask_hbm.at[idx_v.at[c]], mask_v.at[dst], sem)]
            for k in range(NCOMP):
                chunk_descs.append(pltpu.async_copy(
                    vl_hbm.at[idx9_v.at[k * nch + c]], gt_v.at[k, dst], sem))
            descs.append(chunk_descs)

        for dsc in xyz_descs:
            dsc.wait()

        zeros = jnp.zeros((L,), jnp.float32)

        def body(c, g, carry):
            num, den = carry
            sl = pl.ds(c * CH + g * L, L)
            sx = [seed_v[k, sl] for k in range(3)]
            vx = [vote_v[k, sl] for k in range(3)]
            d = None
            for j in range(GTF):
                dj = None
                for k in range(3):
                    t = jnp.abs(vx[k] - (gt_v[3 * j + k, sl] + sx[k]))
                    dj = t if dj is None else dj + t
                d = dj if d is None else jnp.minimum(d, dj)
            mf = mask_v[sl].astype(jnp.float32)
            return num + d * mf, den + mf

        acc = (zeros, zeros)
        for c in range(nch):
            for dsc in descs[c]:
                dsc.wait()
            acc = lax.fori_loop(
                0, CH // L, functools.partial(body, c), acc)
        num, den = acc

        accn_v[...] = num
        accd_v[...] = den
        pltpu.sync_copy(accn_v, num_hbm.at[wid])
        pltpu.sync_copy(accd_v, den_hbm.at[wid])

    return sc_kernel


def kernel(seed_xyz, vote_xyz, seed_inds, vote_label_mask, vote_label):
    B, S, _ = seed_xyz.shape
    P = vote_label.shape[1]
    N = B * S

    idx_g = (seed_inds.astype(jnp.int32)
             + (jnp.arange(B, dtype=jnp.int32) * P)[:, None])
    idx_g = idx_g.reshape(N // CH, CH)
    seed_t = jnp.transpose(seed_xyz, (2, 0, 1)).reshape(3, N)
    vote_t = jnp.transpose(vote_xyz, (2, 0, 1)).reshape(3, N)
    vl_t = jnp.transpose(vote_label, (2, 0, 1)).reshape(NCOMP * B * P)
    mask_flat = vote_label_mask.astype(jnp.int32).reshape(B * P)

    sc = _make_sc_kernel(B, S, P)
    num, den = sc(idx_g, seed_t, vote_t, vl_t, mask_flat)
    return jnp.sum(num) / (jnp.sum(den) + 1e-6)
